# baseline (speedup 1.0000x reference)
import functools

import jax
import jax.numpy as jnp
from jax import lax
from jax.experimental import pallas as pl
from jax.experimental.pallas import tpu as pltpu
from jax.experimental.pallas import tpu_sc as plsc

_B = 1024
_S = 200
_D = 64
_V = 1000000
_NW = 32
_SPW = _B // _NW
_L = 16
_TCB = 512
_SPLITS = ((0, 128), (128, 72))


_GDN = lax.GatherDimensionNumbers(
    offset_dims=(), collapsed_slice_dims=(0,), start_index_map=(0,))


def _shuffle(v, p):
    return lax.gather(v, p[:, None], _GDN, slice_sizes=(1,),
                      mode=lax.GatherScatterMode.PROMISE_IN_BOUNDS)


def _lanesum(v, perms):
    for p in perms:
        v = v + _shuffle(v, p)
    return v


def _rsqrt16(a):
    ai = lax.bitcast_convert_type(a, jnp.int32)
    yi = jnp.int32(0x5F3759DF) - lax.shift_right_arithmetic(ai, jnp.int32(1))
    y = lax.bitcast_convert_type(yi, jnp.float32)
    h = a * jnp.float32(0.5)
    for _ in range(3):
        y = y * (jnp.float32(1.5) - h * y * y)
    return y


def _tc_transpose(table_t):
    def body(x_ref, o_ref):
        o_ref[:, pl.ds(0, _D)] = jnp.transpose(x_ref[...], (1, 0))

    n_blocks = (_V + _TCB - 1) // _TCB
    return pl.pallas_call(
        body,
        grid=(n_blocks,),
        in_specs=[pl.BlockSpec((_D, _TCB), lambda i: (0, i))],
        out_specs=pl.BlockSpec((_TCB, 2 * _D), lambda i: (i, 0)),
        out_shape=jax.ShapeDtypeStruct((_V, 2 * _D), jnp.float32),
    )(table_t)


def _sc_kernel(idx_hbm, table_hbm, gamma_hbm, beta_hbm, pe_hbm, out_hbm,
               idx_v, emb_v, out_v, pe_v, g_v, b_v,
               gsem0, gsem1, osem0, osem1):
    wid = lax.axis_index("s") * 2 + lax.axis_index("c")
    rbase = wid * _SPW * _S
    obase = wid * _SPW * (_S // 2)

    pltpu.sync_copy(pe_hbm, pe_v)
    pltpu.sync_copy(gamma_hbm, g_v)
    pltpu.sync_copy(beta_hbm, b_v)
    pltpu.sync_copy(idx_hbm.at[pl.ds(rbase, _SPW * _S)], idx_v)

    g = [g_v[pl.ds(j * _L, _L)] for j in range(4)]
    b = [b_v[pl.ds(j * _L, _L)] for j in range(4)]
    inv_d = jnp.float32(1.0 / _D)
    lane = lax.iota(jnp.int32, _L)
    perms = [lax.bitwise_xor(lane, jnp.int32(k)) for k in (8, 4, 2, 1)]

    emb0 = emb_v.at[0]
    emb1 = emb_v.at[1]
    out0 = out_v.at[0]
    out1 = out_v.at[1]

    def gather_start(ci, emb_b, gsem):
        for o, n in _SPLITS:
            pltpu.make_async_copy(
                table_hbm.at[idx_v.at[pl.ds(ci * _S + o, n)]],
                emb_b.at[pl.ds(o, n)], gsem).start()

    def gather_wait(emb_b, gsem):
        for o, n in _SPLITS:
            pltpu.make_async_copy(
                table_hbm.at[idx_v.at[pl.ds(o, n)]],
                emb_b.at[pl.ds(o, n)], gsem).wait()

    def out_start(ci, out_b, osem):
        pltpu.make_async_copy(
            out_b, out_hbm.at[pl.ds(obase + ci * (_S // 2), _S // 2)],
            osem).start()

    def out_wait(out_b, osem):
        pltpu.make_async_copy(
            out_b, out_hbm.at[pl.ds(0, _S // 2)], osem).wait()

    def compute(emb_b, out_b):
        def row_body(r, rcarry):
            q = lax.shift_right_logical(r, 1)
            o = (r & jnp.int32(1)) * jnp.int32(_D)
            x = [emb_b[r, pl.ds(j * _L, _L)] + pe_v[r, pl.ds(j * _L, _L)]
                 for j in range(4)]
            s1v = (x[0] + x[1]) + (x[2] + x[3])
            s2v = ((x[0] * x[0] + x[1] * x[1])
                   + (x[2] * x[2] + x[3] * x[3]))
            m = _lanesum(s1v, perms) * inv_d
            ex2 = _lanesum(s2v, perms) * inv_d
            var = ex2 - m * m
            r_std = _rsqrt16(var + jnp.float32(1e-5))
            for j in range(4):
                out_b[q, pl.ds(o + j * _L, _L)] = (
                    (x[j] - m) * r_std * g[j] + b[j])
            return rcarry

        lax.fori_loop(0, _S, row_body, 0, unroll=8)

    gather_start(0, emb0, gsem0)

    def body(i, carry):
        c0 = 2 * i
        c1 = c0 + 1

        @pl.when(i >= 1)
        def _():
            out_wait(out1, osem1)

        gather_start(c1, emb1, gsem1)
        gather_wait(emb0, gsem0)
        compute(emb0, out0)
        out_start(c0, out0, osem0)
        gather_wait(emb1, gsem1)
        compute(emb1, out1)
        out_wait(out0, osem0)

        @pl.when(i <= _SPW // 2 - 2)
        def _():
            gather_start(c0 + 2, emb0, gsem0)

        out_start(c1, out1, osem1)
        return carry

    lax.fori_loop(0, _SPW // 2, body, 0)
    out_wait(out1, osem1)


def kernel(token_ids, table, gamma, beta, pe):
    idx_flat = token_ids.reshape(_B * _S)
    table_pad = _tc_transpose(table.T)

    mesh = plsc.VectorSubcoreMesh(core_axis_name="c", subcore_axis_name="s")
    run = functools.partial(
        pl.kernel,
        mesh=mesh,
        compiler_params=pltpu.CompilerParams(use_tc_tiling_on_sc=False),
        out_type=jax.ShapeDtypeStruct((_B * _S // 2, 128), jnp.float32),
        scratch_types=[
            pltpu.VMEM((_SPW * _S,), jnp.int32),
            pltpu.VMEM((2, _S, 128), jnp.float32),
            pltpu.VMEM((2, _S // 2, 128), jnp.float32),
            pltpu.VMEM((_S, _D), jnp.float32),
            pltpu.VMEM((_D,), jnp.float32),
            pltpu.VMEM((_D,), jnp.float32),
            pltpu.SemaphoreType.DMA,
            pltpu.SemaphoreType.DMA,
            pltpu.SemaphoreType.DMA,
            pltpu.SemaphoreType.DMA,
        ],
    )(_sc_kernel)
    out = run(idx_flat, table_pad, gamma, beta, pe)
    return out.reshape(_B, _S, _D)


# speedup vs baseline: 2.1029x; 2.1029x over previous
import functools

import jax
import jax.numpy as jnp
from jax import lax
from jax.experimental import pallas as pl
from jax.experimental.pallas import tpu as pltpu
from jax.experimental.pallas import tpu_sc as plsc

_B = 1024
_S = 200
_D = 64
_V = 1000000
_NW = 32
_SPW = _B // _NW
_L = 16
_TCB = 8192
_SPLITS = ((0, 128), (128, 72))


_GDN = lax.GatherDimensionNumbers(
    offset_dims=(), collapsed_slice_dims=(0,), start_index_map=(0,))


def _shuffle(v, p):
    return lax.gather(v, p[:, None], _GDN, slice_sizes=(1,),
                      mode=lax.GatherScatterMode.PROMISE_IN_BOUNDS)


def _lanesum(v, perms):
    for p in perms:
        v = v + _shuffle(v, p)
    return v


def _rsqrt16(a):
    ai = lax.bitcast_convert_type(a, jnp.int32)
    yi = jnp.int32(0x5F3759DF) - lax.shift_right_arithmetic(ai, jnp.int32(1))
    y = lax.bitcast_convert_type(yi, jnp.float32)
    h = a * jnp.float32(0.5)
    for _ in range(3):
        y = y * (jnp.float32(1.5) - h * y * y)
    return y


def _tc_transpose(table_t):
    def body(x_ref, o_ref):
        t = jnp.transpose(x_ref[...], (1, 0))
        o_ref[...] = jnp.concatenate([t, t], axis=1)

    n_blocks = (_V + _TCB - 1) // _TCB
    return pl.pallas_call(
        body,
        grid=(n_blocks,),
        in_specs=[pl.BlockSpec((_D, _TCB), lambda i: (0, i))],
        out_specs=pl.BlockSpec((_TCB, 2 * _D), lambda i: (i, 0)),
        out_shape=jax.ShapeDtypeStruct((_V, 2 * _D), jnp.float32),
    )(table_t)


def _sc_kernel(idx_hbm, table_hbm, gamma_hbm, beta_hbm, pe_hbm, out_hbm,
               idx_v, emb_v, out_v, pe_v, g_v, b_v,
               gsem0, gsem1, osem0, osem1):
    wid = lax.axis_index("s") * 2 + lax.axis_index("c")
    rbase = wid * _SPW * _S
    obase = wid * _SPW * (_S // 2)

    pltpu.sync_copy(pe_hbm, pe_v)
    pltpu.sync_copy(gamma_hbm, g_v)
    pltpu.sync_copy(beta_hbm, b_v)
    pltpu.sync_copy(idx_hbm.at[pl.ds(rbase, _SPW * _S)], idx_v)

    g = [g_v[pl.ds(j * _L, _L)] for j in range(4)]
    b = [b_v[pl.ds(j * _L, _L)] for j in range(4)]
    inv_d = jnp.float32(1.0 / _D)
    lane = lax.iota(jnp.int32, _L)
    perms = [lax.bitwise_xor(lane, jnp.int32(k)) for k in (8, 4, 2, 1)]

    emb0 = emb_v.at[0]
    emb1 = emb_v.at[1]
    out0 = out_v.at[0]
    out1 = out_v.at[1]

    def gather_start(ci, emb_b, gsem):
        for o, n in _SPLITS:
            pltpu.make_async_copy(
                table_hbm.at[idx_v.at[pl.ds(ci * _S + o, n)]],
                emb_b.at[pl.ds(o, n)], gsem).start()

    def gather_wait(emb_b, gsem):
        for o, n in _SPLITS:
            pltpu.make_async_copy(
                table_hbm.at[idx_v.at[pl.ds(o, n)]],
                emb_b.at[pl.ds(o, n)], gsem).wait()

    def out_start(ci, out_b, osem):
        pltpu.make_async_copy(
            out_b, out_hbm.at[pl.ds(obase + ci * (_S // 2), _S // 2)],
            osem).start()

    def out_wait(out_b, osem):
        pltpu.make_async_copy(
            out_b, out_hbm.at[pl.ds(0, _S // 2)], osem).wait()

    def compute(emb_b, out_b):
        def row_body(r, rcarry):
            q = lax.shift_right_logical(r, 1)
            o = (r & jnp.int32(1)) * jnp.int32(_D)
            x = [emb_b[r, pl.ds(j * _L, _L)] + pe_v[r, pl.ds(j * _L, _L)]
                 for j in range(4)]
            s1v = (x[0] + x[1]) + (x[2] + x[3])
            s2v = ((x[0] * x[0] + x[1] * x[1])
                   + (x[2] * x[2] + x[3] * x[3]))
            m = _lanesum(s1v, perms) * inv_d
            ex2 = _lanesum(s2v, perms) * inv_d
            var = ex2 - m * m
            r_std = _rsqrt16(var + jnp.float32(1e-5))
            for j in range(4):
                out_b[q, pl.ds(o + j * _L, _L)] = (
                    (x[j] - m) * r_std * g[j] + b[j])
            return rcarry

        lax.fori_loop(0, _S, row_body, 0, unroll=8)

    gather_start(0, emb0, gsem0)

    def body(i, carry):
        c0 = 2 * i
        c1 = c0 + 1

        @pl.when(i >= 1)
        def _():
            out_wait(out1, osem1)

        gather_start(c1, emb1, gsem1)
        gather_wait(emb0, gsem0)
        compute(emb0, out0)
        out_start(c0, out0, osem0)
        gather_wait(emb1, gsem1)
        compute(emb1, out1)
        out_wait(out0, osem0)

        @pl.when(i <= _SPW // 2 - 2)
        def _():
            gather_start(c0 + 2, emb0, gsem0)

        out_start(c1, out1, osem1)
        return carry

    lax.fori_loop(0, _SPW // 2, body, 0)
    out_wait(out1, osem1)


def kernel(token_ids, table, gamma, beta, pe):
    idx_flat = token_ids.reshape(_B * _S)
    table_pad = _tc_transpose(table.T)

    mesh = plsc.VectorSubcoreMesh(core_axis_name="c", subcore_axis_name="s")
    run = functools.partial(
        pl.kernel,
        mesh=mesh,
        compiler_params=pltpu.CompilerParams(use_tc_tiling_on_sc=False),
        out_type=jax.ShapeDtypeStruct((_B * _S // 2, 128), jnp.float32),
        scratch_types=[
            pltpu.VMEM((_SPW * _S,), jnp.int32),
            pltpu.VMEM((2, _S, 128), jnp.float32),
            pltpu.VMEM((2, _S // 2, 128), jnp.float32),
            pltpu.VMEM((_S, _D), jnp.float32),
            pltpu.VMEM((_D,), jnp.float32),
            pltpu.VMEM((_D,), jnp.float32),
            pltpu.SemaphoreType.DMA,
            pltpu.SemaphoreType.DMA,
            pltpu.SemaphoreType.DMA,
            pltpu.SemaphoreType.DMA,
        ],
    )(_sc_kernel)
    out = run(idx_flat, table_pad, gamma, beta, pe)
    return out.reshape(_B, _S, _D)


# speedup vs baseline: 2.4149x; 1.1484x over previous
import functools

import jax
import jax.numpy as jnp
from jax import lax
from jax.experimental import pallas as pl
from jax.experimental.pallas import tpu as pltpu
from jax.experimental.pallas import tpu_sc as plsc

_B = 1024
_S = 200
_D = 64
_V = 1000000
_NW = 32
_SPW = _B // _NW
_L = 16
_TCB = 16384
_SPLITS = ((0, 128), (128, 72))


_GDN = lax.GatherDimensionNumbers(
    offset_dims=(), collapsed_slice_dims=(0,), start_index_map=(0,))


def _shuffle(v, p):
    return lax.gather(v, p[:, None], _GDN, slice_sizes=(1,),
                      mode=lax.GatherScatterMode.PROMISE_IN_BOUNDS)


def _lanesum(v, perms):
    for p in perms:
        v = v + _shuffle(v, p)
    return v


def _rsqrt16(a):
    ai = lax.bitcast_convert_type(a, jnp.int32)
    yi = jnp.int32(0x5F3759DF) - lax.shift_right_arithmetic(ai, jnp.int32(1))
    y = lax.bitcast_convert_type(yi, jnp.float32)
    h = a * jnp.float32(0.5)
    for _ in range(2):
        y = y * (jnp.float32(1.5) - h * y * y)
    return y


def _tc_transpose(table_t):
    def body(x_ref, o_ref):
        o_ref[:, pl.ds(0, _D)] = jnp.transpose(x_ref[...], (1, 0))

    n_blocks = (_V + _TCB - 1) // _TCB
    return pl.pallas_call(
        body,
        grid=(n_blocks,),
        in_specs=[pl.BlockSpec((_D, _TCB), lambda i: (0, i))],
        out_specs=pl.BlockSpec((_TCB, 2 * _D), lambda i: (i, 0)),
        out_shape=jax.ShapeDtypeStruct((_V, 2 * _D), jnp.float32),
    )(table_t)


def _sc_kernel(idx_hbm, table_hbm, gamma_hbm, beta_hbm, pe_hbm, out_hbm,
               idx_v, emb_v, out_v, pe_v, g_v, b_v,
               gsem0, gsem1, osem0, osem1):
    wid = lax.axis_index("s") * 2 + lax.axis_index("c")
    rbase = wid * _SPW * _S
    obase = wid * _SPW * (_S // 2)

    pltpu.sync_copy(pe_hbm, pe_v)
    pltpu.sync_copy(gamma_hbm, g_v)
    pltpu.sync_copy(beta_hbm, b_v)
    pltpu.sync_copy(idx_hbm.at[pl.ds(rbase, _SPW * _S)], idx_v)

    g = [g_v[pl.ds(j * _L, _L)] for j in range(4)]
    b = [b_v[pl.ds(j * _L, _L)] for j in range(4)]
    inv_d = jnp.float32(1.0 / _D)
    lane = lax.iota(jnp.int32, _L)
    perms = [lax.bitwise_xor(lane, jnp.int32(k)) for k in (8, 4, 2, 1)]

    emb0 = emb_v.at[0]
    emb1 = emb_v.at[1]
    out0 = out_v.at[0]
    out1 = out_v.at[1]

    def gather_start(ci, emb_b, gsem):
        for o, n in _SPLITS:
            pltpu.make_async_copy(
                table_hbm.at[idx_v.at[pl.ds(ci * _S + o, n)]],
                emb_b.at[pl.ds(o, n)], gsem).start()

    def gather_wait(emb_b, gsem):
        for o, n in _SPLITS:
            pltpu.make_async_copy(
                table_hbm.at[idx_v.at[pl.ds(o, n)]],
                emb_b.at[pl.ds(o, n)], gsem).wait()

    def out_start(ci, out_b, osem):
        pltpu.make_async_copy(
            out_b, out_hbm.at[pl.ds(obase + ci * (_S // 2), _S // 2)],
            osem).start()

    def out_wait(out_b, osem):
        pltpu.make_async_copy(
            out_b, out_hbm.at[pl.ds(0, _S // 2)], osem).wait()

    def compute(emb_b, out_b):
        def row_body(r, rcarry):
            q = lax.shift_right_logical(r, 1)
            o = (r & jnp.int32(1)) * jnp.int32(_D)
            x = [emb_b[r, pl.ds(j * _L, _L)] + pe_v[r, pl.ds(j * _L, _L)]
                 for j in range(4)]
            s1v = (x[0] + x[1]) + (x[2] + x[3])
            s2v = ((x[0] * x[0] + x[1] * x[1])
                   + (x[2] * x[2] + x[3] * x[3]))
            m = _lanesum(s1v, perms) * inv_d
            ex2 = _lanesum(s2v, perms) * inv_d
            var = ex2 - m * m
            r_std = _rsqrt16(var + jnp.float32(1e-5))
            for j in range(4):
                out_b[q, pl.ds(o + j * _L, _L)] = (
                    (x[j] - m) * r_std * g[j] + b[j])
            return rcarry

        lax.fori_loop(0, _S, row_body, 0, unroll=8)

    gather_start(0, emb0, gsem0)

    def body(i, carry):
        c0 = 2 * i
        c1 = c0 + 1

        @pl.when(i >= 1)
        def _():
            out_wait(out1, osem1)

        gather_start(c1, emb1, gsem1)
        gather_wait(emb0, gsem0)
        compute(emb0, out0)
        out_start(c0, out0, osem0)
        gather_wait(emb1, gsem1)
        compute(emb1, out1)
        out_wait(out0, osem0)

        @pl.when(i <= _SPW // 2 - 2)
        def _():
            gather_start(c0 + 2, emb0, gsem0)

        out_start(c1, out1, osem1)
        return carry

    lax.fori_loop(0, _SPW // 2, body, 0)
    out_wait(out1, osem1)


def kernel(token_ids, table, gamma, beta, pe):
    idx_flat = token_ids.reshape(_B * _S)
    table_pad = _tc_transpose(table.T)

    mesh = plsc.VectorSubcoreMesh(core_axis_name="c", subcore_axis_name="s")
    run = functools.partial(
        pl.kernel,
        mesh=mesh,
        compiler_params=pltpu.CompilerParams(use_tc_tiling_on_sc=False),
        out_type=jax.ShapeDtypeStruct((_B * _S // 2, 128), jnp.float32),
        scratch_types=[
            pltpu.VMEM((_SPW * _S,), jnp.int32),
            pltpu.VMEM((2, _S, 128), jnp.float32),
            pltpu.VMEM((2, _S // 2, 128), jnp.float32),
            pltpu.VMEM((_S, _D), jnp.float32),
            pltpu.VMEM((_D,), jnp.float32),
            pltpu.VMEM((_D,), jnp.float32),
            pltpu.SemaphoreType.DMA,
            pltpu.SemaphoreType.DMA,
            pltpu.SemaphoreType.DMA,
            pltpu.SemaphoreType.DMA,
        ],
    )(_sc_kernel)
    out = run(idx_flat, table_pad, gamma, beta, pe)
    return out.reshape(_B, _S, _D)


# speedup vs baseline: 3.2129x; 1.3304x over previous
import functools

import jax
import jax.numpy as jnp
from jax import lax
from jax.experimental import pallas as pl
from jax.experimental.pallas import tpu as pltpu
from jax.experimental.pallas import tpu_sc as plsc

_B = 1024
_S = 200
_D = 64
_V = 1000000
_NW = 32
_TPW = _B * _S // _NW
_C = 320
_NCH = _TPW // _C
_SPLITS = tuple((o, min(128, _C - o)) for o in range(0, _C, 128))
_TCB = 16384
_LNR = 1600


def _tc_transpose(table_t):
    def body(x_ref, o_ref):
        o_ref[:, pl.ds(0, _D)] = jnp.transpose(x_ref[...], (1, 0))

    n_blocks = (_V + _TCB - 1) // _TCB
    return pl.pallas_call(
        body,
        grid=(n_blocks,),
        in_specs=[pl.BlockSpec((_D, _TCB), lambda i: (0, i))],
        out_specs=pl.BlockSpec((_TCB, 2 * _D), lambda i: (i, 0)),
        out_shape=jax.ShapeDtypeStruct((_V, 2 * _D), jnp.float32),
    )(table_t)


def _sc_gather(idx_hbm, table_hbm, out_hbm, idx_v, emb_v,
               gsem0, gsem1, osem0, osem1):
    wid = lax.axis_index("s") * 2 + lax.axis_index("c")
    tbase = wid * _TPW

    pltpu.sync_copy(idx_hbm.at[pl.ds(tbase, _TPW)], idx_v)

    emb0 = emb_v.at[0]
    emb1 = emb_v.at[1]

    def gather_start(ci, emb_b, gsem):
        for o, n in _SPLITS:
            pltpu.make_async_copy(
                table_hbm.at[idx_v.at[pl.ds(ci * _C + o, n)]],
                emb_b.at[pl.ds(o, n)], gsem).start()

    def gather_wait(emb_b, gsem):
        for o, n in _SPLITS:
            pltpu.make_async_copy(
                table_hbm.at[idx_v.at[pl.ds(o, n)]],
                emb_b.at[pl.ds(o, n)], gsem).wait()

    def out_start(ci, emb_b, osem):
        pltpu.make_async_copy(
            emb_b, out_hbm.at[pl.ds(tbase + ci * _C, _C)], osem).start()

    def out_wait(emb_b, osem):
        pltpu.make_async_copy(
            emb_b, out_hbm.at[pl.ds(0, _C)], osem).wait()

    gather_start(0, emb0, gsem0)

    def body(i, carry):
        c0 = 2 * i
        c1 = c0 + 1

        @pl.when(i >= 1)
        def _():
            out_wait(emb1, osem1)

        gather_start(c1, emb1, gsem1)
        gather_wait(emb0, gsem0)
        out_start(c0, emb0, osem0)
        gather_wait(emb1, gsem1)
        out_wait(emb0, osem0)

        @pl.when(i <= _NCH // 2 - 2)
        def _():
            gather_start(c0 + 2, emb0, gsem0)

        out_start(c1, emb1, osem1)
        return carry

    lax.fori_loop(0, _NCH // 2, body, 0)
    out_wait(emb1, osem1)


def _tc_ln(x_ref, pe_ref, g_ref, b_ref, o_ref):
    x = x_ref[:, pl.ds(0, _D)] + pe_ref[...]
    g = g_ref[...][None, :]
    b = b_ref[...][None, :]
    inv_d = jnp.float32(1.0 / _D)
    m = jnp.sum(x, axis=1, keepdims=True) * inv_d
    ex2 = jnp.sum(x * x, axis=1, keepdims=True) * inv_d
    r = lax.rsqrt(ex2 - m * m + jnp.float32(1e-5))
    o_ref[...] = (x - m) * r * g + b


def kernel(token_ids, table, gamma, beta, pe):
    idx_flat = token_ids.reshape(_B * _S)
    table_pad = _tc_transpose(table.T)

    mesh = plsc.VectorSubcoreMesh(core_axis_name="c", subcore_axis_name="s")
    run = functools.partial(
        pl.kernel,
        mesh=mesh,
        compiler_params=pltpu.CompilerParams(use_tc_tiling_on_sc=False),
        out_type=jax.ShapeDtypeStruct((_B * _S, 2 * _D), jnp.float32),
        scratch_types=[
            pltpu.VMEM((_TPW,), jnp.int32),
            pltpu.VMEM((2, _C, 2 * _D), jnp.float32),
            pltpu.SemaphoreType.DMA,
            pltpu.SemaphoreType.DMA,
            pltpu.SemaphoreType.DMA,
            pltpu.SemaphoreType.DMA,
        ],
    )(_sc_gather)
    gathered = run(idx_flat, table_pad)

    pe_tiled = jnp.tile(pe, (_LNR // _S, 1))
    n_blocks = (_B * _S) // _LNR
    out = pl.pallas_call(
        _tc_ln,
        grid=(n_blocks,),
        in_specs=[
            pl.BlockSpec((_LNR, 2 * _D), lambda i: (i, 0)),
            pl.BlockSpec((_LNR, _D), lambda i: (0, 0)),
            pl.BlockSpec((_D,), lambda i: (0,)),
            pl.BlockSpec((_D,), lambda i: (0,)),
        ],
        out_specs=pl.BlockSpec((_LNR, _D), lambda i: (i, 0)),
        out_shape=jax.ShapeDtypeStruct((_B * _S, _D), jnp.float32),
    )(gathered, pe_tiled, gamma, beta)
    return out.reshape(_B, _S, _D)
